# baseline (speedup 1.0000x reference)
import functools

import numpy as np
import jax
import jax.numpy as jnp
from jax import lax
from jax.experimental import pallas as pl
from jax.experimental.pallas import tpu as pltpu
from jax.experimental.pallas import tpu_sc as plsc

_DEEP = 256
_TYPE = 10
_PARAM = 12
_F = _DEEP + _TYPE + _PARAM

_api_lists = [[0], [0, 1], [1, 2], [3], [4, 5], [6], [7, 8], [9], [10], [11]]
_API_NP = np.zeros((_TYPE, _PARAM), dtype=np.float32)
for _t, _lst in enumerate(_api_lists):
    for _p in _lst:
        _API_NP[_t, _p] = 1.0

_TC_B = 16
_ROWS = 2048
_BB = 1
_BB2 = 4

_NW = 32
_CH = 64


def _tc_body(x_ref, t_ref, api_ref, o_ref):
    i = pl.program_id(0)
    j = pl.program_id(1)
    x = x_ref[...].reshape(_BB * _ROWS, _F)
    t = t_ref[...].reshape(_BB * _ROWS, _F)

    pad = t[:, _DEEP:_DEEP + 1] == -1.0
    validf = jnp.where(pad, 0.0, 1.0)[:, 0]

    xs = x[:, :_DEEP]
    ts = t[:, :_DEEP]
    ds = jnp.where(pad, -100.0 - ts, xs - ts)
    s_svg = jnp.sum(ds * ds)

    xt = x[:, _DEEP:_DEEP + _TYPE]
    tt = t[:, _DEEP:_DEEP + _TYPE]
    m = jnp.max(xt, axis=1, keepdims=True)
    lse = m[:, 0] + jnp.log(jnp.sum(jnp.exp(xt - m), axis=1))
    picked = jnp.sum(xt * tt, axis=1)
    s_type = jnp.sum((lse - picked) * validf)
    cnt = jnp.sum(validf)

    xp = x[:, _DEEP + _TYPE:]
    tp = t[:, _DEEP + _TYPE:]
    copy = lax.dot(tt, api_ref[...], preferred_element_type=jnp.float32) > 0.5
    dp = jnp.where(pad, -100.0 - tp, jnp.where(copy, 0.0, xp - tp))
    s_param = jnp.sum(dp * dp)

    @pl.when((i == 0) & (j == 0))
    def _init():
        o_ref[0] = 0.0
        o_ref[1] = 0.0
        o_ref[2] = 0.0
        o_ref[3] = 0.0

    o_ref[0] += s_svg
    o_ref[1] += s_type
    o_ref[2] += cnt
    o_ref[3] += s_param


def _tc_sums(input, target, nb):
    return pl.pallas_call(
        _tc_body,
        grid=(nb // _BB, input.shape[1] // _ROWS),
        in_specs=[
            pl.BlockSpec((_BB, _ROWS, _F), lambda i, j: (i, j, 0)),
            pl.BlockSpec((_BB, _ROWS, _F), lambda i, j: (i, j, 0)),
            pl.BlockSpec((_TYPE, _PARAM), lambda i, j: (0, 0)),
        ],
        out_specs=pl.BlockSpec(memory_space=pltpu.SMEM),
        out_shape=jax.ShapeDtypeStruct((4,), jnp.float32),
    )(input, target, jnp.asarray(_API_NP))


def _corr_body(x_ref, t_ref, api_ref, o_ref):
    i = pl.program_id(0)
    x = x_ref[...].reshape(_BB2 * _ROWS, 128)
    t = t_ref[...].reshape(_BB2 * _ROWS, 128)

    pad = t[:, 0:1] == -1.0
    validf = jnp.where(pad, 0.0, 1.0)[:, 0]

    xt = x[:, :_TYPE]
    tt = t[:, :_TYPE]
    m = jnp.max(xt, axis=1, keepdims=True)
    lse = m[:, 0] + jnp.log(jnp.sum(jnp.exp(xt - m), axis=1))
    picked = jnp.sum(xt * tt, axis=1)
    s_type = jnp.sum((lse - picked) * validf)
    cnt = jnp.sum(validf)

    xp = x[:, _TYPE:_TYPE + _PARAM]
    tp = t[:, _TYPE:_TYPE + _PARAM]
    copy = lax.dot(tt, api_ref[...], preferred_element_type=jnp.float32) > 0.5
    dp = jnp.where(pad, -100.0 - tp, jnp.where(copy, 0.0, xp - tp))
    s_param = jnp.sum(dp * dp)

    @pl.when(i == 0)
    def _init():
        o_ref[0] = 0.0
        o_ref[1] = 0.0
        o_ref[2] = 0.0

    o_ref[0] += s_type
    o_ref[1] += cnt
    o_ref[2] += s_param


def _corr_sums(input, target, b0, nb):
    i0 = b0 // _BB2
    return pl.pallas_call(
        _corr_body,
        grid=(nb // _BB2,),
        in_specs=[
            pl.BlockSpec((_BB2, _ROWS, 128), lambda i: (i0 + i, 0, 2)),
            pl.BlockSpec((_BB2, _ROWS, 128), lambda i: (i0 + i, 0, 2)),
            pl.BlockSpec((_TYPE, _PARAM), lambda i: (0, 0)),
        ],
        out_specs=pl.BlockSpec(memory_space=pltpu.SMEM),
        out_shape=jax.ShapeDtypeStruct((3,), jnp.float32),
    )(input, target, jnp.asarray(_API_NP))


def _sc_make(nb):
    mesh = plsc.VectorSubcoreMesh(core_axis_name="c", subcore_axis_name="s",
                                  num_cores=2, num_subcores=16)

    @functools.partial(
        pl.kernel,
        out_type=jax.ShapeDtypeStruct((_NW, 16), jnp.float32),
        mesh=mesh,
        compiler_params=pltpu.CompilerParams(use_tc_tiling_on_sc=True),
        scratch_types=[
            pltpu.VMEM((2, _CH, _F), jnp.float32),
            pltpu.VMEM((2, _CH, _F), jnp.float32),
            pltpu.VMEM((2, 128), jnp.float32),
            pltpu.VMEM((16,), jnp.float32),
            pltpu.SemaphoreType.DMA,
            pltpu.SemaphoreType.DMA,
            pltpu.SemaphoreType.DMA,
            pltpu.SemaphoreType.DMA,
            pltpu.SemaphoreType.DMA,
            pltpu.SemaphoreType.DMA,
        ],
    )
    def run(x_hbm, t_hbm, m_hbm, o_hbm, xb, tb, mb, ostage,
            sx0, sx1, st0, st1, sm0, sm1):
        b0 = 0
        cid = lax.axis_index("c")
        sid = lax.axis_index("s")
        wid = sid * 2 + cid
        r0 = wid * _CH
        sx = (sx0, sx1)
        st = (st0, st1)
        sm = (sm0, sm1)
        iota = lax.iota(jnp.int32, 16)

        def issue(k, p):
            pltpu.async_copy(x_hbm.at[pl.ds((b0 + k) * 2048 + r0, _CH), :],
                             xb.at[p], sx[p])
            pltpu.async_copy(t_hbm.at[pl.ds((b0 + k) * 2048 + r0, _CH), :],
                             tb.at[p], st[p])
            pltpu.async_copy(m_hbm.at[_TC_B + b0 + k, wid, :],
                             mb.at[p], sm[p])

        def wait(k, p):
            pltpu.make_async_copy(
                x_hbm.at[pl.ds((b0 + k) * 2048 + r0, _CH), :],
                xb.at[p], sx[p]).wait()
            pltpu.make_async_copy(
                t_hbm.at[pl.ds((b0 + k) * 2048 + r0, _CH), :],
                tb.at[p], st[p]).wait()
            pltpu.make_async_copy(
                m_hbm.at[_TC_B + b0 + k, wid, :],
                mb.at[p], sm[p]).wait()

        def process(p, acc):
            xc = xb.at[p]
            tc_ = tb.at[p]
            mc = mb.at[p]

            def svg_row(r, a):
                padr = mc[pl.ds(r, 16)][0] == 1.0
                for kk in range(_DEEP // 16):
                    xk = xc[r, pl.ds(16 * kk, 16)]
                    tk = tc_[r, pl.ds(16 * kk, 16)]
                    d = jnp.where(padr, 100.0 + tk, xk - tk)
                    a = a + d * d
                return a

            return lax.fori_loop(0, _CH, svg_row, acc)

        zero = iota.astype(jnp.float32) * 0.0
        issue(0, 0)
        issue(1, 1)

        def outer(kk, acc):
            k0 = 2 * kk
            wait(k0, 0)
            acc = process(0, acc)

            @pl.when(k0 + 2 < nb)
            def _():
                issue(k0 + 2, 0)

            wait(k0 + 1, 1)
            acc = process(1, acc)

            @pl.when(k0 + 3 < nb)
            def _():
                issue(k0 + 3, 1)

            return acc

        acc = lax.fori_loop(0, nb // 2, outer, zero)
        ostage[...] = acc
        pltpu.sync_copy(ostage, o_hbm.at[wid])

    return run


def kernel(input, target, target_padding_mask):
    b, s, _ = input.shape
    n = b * s
    mask32 = jnp.pad(
        target_padding_mask.reshape(b, _NW, _CH).astype(jnp.float32),
        ((0, 0), (0, 0), (0, 128 - _CH)))
    xr = input[_TC_B:].reshape((b - _TC_B) * s, _F)
    tr = target[_TC_B:].reshape((b - _TC_B) * s, _F)
    sc = _sc_make(b - _TC_B)(xr, tr, mask32)
    tc = _tc_sums(input, target, _TC_B)
    corr = _corr_sums(input, target, _TC_B, b - _TC_B)
    sc_svg = jnp.sum(sc)

    cnt = jnp.maximum(tc[2] + corr[1], 1.0)
    loss = (10.0 * (tc[0] + sc_svg) / (n * _DEEP)
            + (tc[3] + corr[2]) / (n * _PARAM)
            + 0.1 * (tc[1] + corr[0]) / cnt)
    return loss


# speedup vs baseline: 1.7959x; 1.7959x over previous
import functools

import numpy as np
import jax
import jax.numpy as jnp
from jax import lax
from jax.experimental import pallas as pl
from jax.experimental.pallas import tpu as pltpu
from jax.experimental.pallas import tpu_sc as plsc

_DEEP = 256
_TYPE = 10
_PARAM = 12
_F = _DEEP + _TYPE + _PARAM

_api_lists = [[0], [0, 1], [1, 2], [3], [4, 5], [6], [7, 8], [9], [10], [11]]
_API_NP = np.zeros((_TYPE, _PARAM), dtype=np.float32)
for _t, _lst in enumerate(_api_lists):
    for _p in _lst:
        _API_NP[_t, _p] = 1.0

_TC_B = 0
_ROWS = 2048
_BB = 1
_BB2 = 4

_NW = 32
_CH = 64


def _tc_body(x_ref, t_ref, api_ref, o_ref):
    i = pl.program_id(0)
    j = pl.program_id(1)
    x = x_ref[...].reshape(_BB * _ROWS, _F)
    t = t_ref[...].reshape(_BB * _ROWS, _F)

    pad = t[:, _DEEP:_DEEP + 1] == -1.0
    validf = jnp.where(pad, 0.0, 1.0)[:, 0]

    xs = x[:, :_DEEP]
    ts = t[:, :_DEEP]
    ds = jnp.where(pad, -100.0 - ts, xs - ts)
    s_svg = jnp.sum(ds * ds)

    xt = x[:, _DEEP:_DEEP + _TYPE]
    tt = t[:, _DEEP:_DEEP + _TYPE]
    m = jnp.max(xt, axis=1, keepdims=True)
    lse = m[:, 0] + jnp.log(jnp.sum(jnp.exp(xt - m), axis=1))
    picked = jnp.sum(xt * tt, axis=1)
    s_type = jnp.sum((lse - picked) * validf)
    cnt = jnp.sum(validf)

    xp = x[:, _DEEP + _TYPE:]
    tp = t[:, _DEEP + _TYPE:]
    copy = lax.dot(tt, api_ref[...], preferred_element_type=jnp.float32) > 0.5
    dp = jnp.where(pad, -100.0 - tp, jnp.where(copy, 0.0, xp - tp))
    s_param = jnp.sum(dp * dp)

    @pl.when((i == 0) & (j == 0))
    def _init():
        o_ref[0] = 0.0
        o_ref[1] = 0.0
        o_ref[2] = 0.0
        o_ref[3] = 0.0

    o_ref[0] += s_svg
    o_ref[1] += s_type
    o_ref[2] += cnt
    o_ref[3] += s_param


def _tc_sums(input, target, nb):
    return pl.pallas_call(
        _tc_body,
        grid=(nb // _BB, input.shape[1] // _ROWS),
        in_specs=[
            pl.BlockSpec((_BB, _ROWS, _F), lambda i, j: (i, j, 0)),
            pl.BlockSpec((_BB, _ROWS, _F), lambda i, j: (i, j, 0)),
            pl.BlockSpec((_TYPE, _PARAM), lambda i, j: (0, 0)),
        ],
        out_specs=pl.BlockSpec(memory_space=pltpu.SMEM),
        out_shape=jax.ShapeDtypeStruct((4,), jnp.float32),
    )(input, target, jnp.asarray(_API_NP))


def _corr_body(x_ref, t_ref, api_ref, o_ref):
    i = pl.program_id(0)
    x = x_ref[...].reshape(_BB2 * _ROWS, 128)
    t = t_ref[...].reshape(_BB2 * _ROWS, 128)

    pad = t[:, 0:1] == -1.0
    validf = jnp.where(pad, 0.0, 1.0)[:, 0]

    xt = x[:, :_TYPE]
    tt = t[:, :_TYPE]
    m = jnp.max(xt, axis=1, keepdims=True)
    lse = m[:, 0] + jnp.log(jnp.sum(jnp.exp(xt - m), axis=1))
    picked = jnp.sum(xt * tt, axis=1)
    s_type = jnp.sum((lse - picked) * validf)
    cnt = jnp.sum(validf)

    xp = x[:, _TYPE:_TYPE + _PARAM]
    tp = t[:, _TYPE:_TYPE + _PARAM]
    copy = lax.dot(tt, api_ref[...], preferred_element_type=jnp.float32) > 0.5
    dp = jnp.where(pad, -100.0 - tp, jnp.where(copy, 0.0, xp - tp))
    s_param = jnp.sum(dp * dp)

    @pl.when(i == 0)
    def _init():
        o_ref[0] = 0.0
        o_ref[1] = 0.0
        o_ref[2] = 0.0

    o_ref[0] += s_type
    o_ref[1] += cnt
    o_ref[2] += s_param


def _corr_sums(input, target, b0, nb):
    i0 = b0 // _BB2
    return pl.pallas_call(
        _corr_body,
        grid=(nb // _BB2,),
        in_specs=[
            pl.BlockSpec((_BB2, _ROWS, 128), lambda i: (i0 + i, 0, 2)),
            pl.BlockSpec((_BB2, _ROWS, 128), lambda i: (i0 + i, 0, 2)),
            pl.BlockSpec((_TYPE, _PARAM), lambda i: (0, 0)),
        ],
        out_specs=pl.BlockSpec(memory_space=pltpu.SMEM),
        out_shape=jax.ShapeDtypeStruct((3,), jnp.float32),
    )(input, target, jnp.asarray(_API_NP))


def _sc_make(b0, nb):
    mesh = plsc.VectorSubcoreMesh(core_axis_name="c", subcore_axis_name="s",
                                  num_cores=2, num_subcores=16)

    @functools.partial(
        pl.kernel,
        out_type=jax.ShapeDtypeStruct((_NW, 16), jnp.float32),
        mesh=mesh,
        compiler_params=pltpu.CompilerParams(use_tc_tiling_on_sc=True),
        scratch_types=[
            pltpu.VMEM((2, _CH, _DEEP), jnp.float32),
            pltpu.VMEM((2, _CH, _DEEP), jnp.float32),
            pltpu.VMEM((2, 128), jnp.float32),
            pltpu.VMEM((16,), jnp.float32),
            pltpu.SemaphoreType.DMA,
            pltpu.SemaphoreType.DMA,
            pltpu.SemaphoreType.DMA,
            pltpu.SemaphoreType.DMA,
            pltpu.SemaphoreType.DMA,
            pltpu.SemaphoreType.DMA,
        ],
    )
    def run(x_hbm, t_hbm, m_hbm, o_hbm, xb, tb, mb, ostage,
            sx0, sx1, st0, st1, sm0, sm1):
        cid = lax.axis_index("c")
        sid = lax.axis_index("s")
        wid = sid * 2 + cid
        r0 = wid * _CH
        sx = (sx0, sx1)
        st = (st0, st1)
        sm = (sm0, sm1)
        iota = lax.iota(jnp.int32, 16)

        def issue(k, p):
            pltpu.async_copy(x_hbm.at[b0 + k, pl.ds(r0, _CH), pl.ds(0, _DEEP)],
                             xb.at[p], sx[p])
            pltpu.async_copy(t_hbm.at[b0 + k, pl.ds(r0, _CH), pl.ds(0, _DEEP)],
                             tb.at[p], st[p])
            pltpu.async_copy(m_hbm.at[b0 + k, wid, :],
                             mb.at[p], sm[p])

        def wait(k, p):
            pltpu.make_async_copy(
                x_hbm.at[b0 + k, pl.ds(r0, _CH), pl.ds(0, _DEEP)],
                xb.at[p], sx[p]).wait()
            pltpu.make_async_copy(
                t_hbm.at[b0 + k, pl.ds(r0, _CH), pl.ds(0, _DEEP)],
                tb.at[p], st[p]).wait()
            pltpu.make_async_copy(
                m_hbm.at[b0 + k, wid, :],
                mb.at[p], sm[p]).wait()

        def process(p, acc):
            xc = xb.at[p]
            tc_ = tb.at[p]
            mc = mb.at[p]

            def svg_row(r, a):
                padr = mc[pl.ds(r, 16)][0] == 1.0
                for kk in range(_DEEP // 16):
                    xk = xc[r, pl.ds(16 * kk, 16)]
                    tk = tc_[r, pl.ds(16 * kk, 16)]
                    d = jnp.where(padr, 100.0 + tk, xk - tk)
                    a = a + d * d
                return a

            return lax.fori_loop(0, _CH, svg_row, acc)

        zero = iota.astype(jnp.float32) * 0.0
        issue(0, 0)
        issue(1, 1)

        def outer(kk, acc):
            k0 = 2 * kk
            wait(k0, 0)
            acc = process(0, acc)

            @pl.when(k0 + 2 < nb)
            def _():
                issue(k0 + 2, 0)

            wait(k0 + 1, 1)
            acc = process(1, acc)

            @pl.when(k0 + 3 < nb)
            def _():
                issue(k0 + 3, 1)

            return acc

        acc = lax.fori_loop(0, nb // 2, outer, zero)
        ostage[...] = acc
        pltpu.sync_copy(ostage, o_hbm.at[wid])

    return run


def kernel(input, target, target_padding_mask):
    b, s, _ = input.shape
    n = b * s
    mask32 = jnp.pad(
        target_padding_mask.reshape(b, _NW, _CH).astype(jnp.float32),
        ((0, 0), (0, 0), (0, 128 - _CH)))
    sc = _sc_make(_TC_B, b - _TC_B)(input, target, mask32)
    if _TC_B:
        tc = _tc_sums(input, target, _TC_B)
    else:
        tc = jnp.zeros((4,), jnp.float32)
    corr = _corr_sums(input, target, _TC_B, b - _TC_B)
    sc_svg = jnp.sum(sc)

    cnt = jnp.maximum(tc[2] + corr[1], 1.0)
    loss = (10.0 * (tc[0] + sc_svg) / (n * _DEEP)
            + (tc[3] + corr[2]) / (n * _PARAM)
            + 0.1 * (tc[1] + corr[0]) / cnt)
    return loss
